# baseline (speedup 1.0000x reference)
import functools

import jax
import jax.numpy as jnp
from jax import lax
from jax.experimental import pallas as pl
from jax.experimental.pallas import tpu as pltpu
from jax.experimental.pallas import tpu_sc as plsc

_NUM_CORES = 2
_NUM_SUBCORES = 16
_NW = _NUM_CORES * _NUM_SUBCORES
_BCHUNK = 16
_SPAD = 56
_DPAD = 128


@functools.partial(jax.jit, static_argnums=(3,))
def _embed(idx, wa, wb, seq):
    NB, SP = idx.shape
    V, DH = wa.shape
    nb_per_w = NB // _NW
    n_chunks = nb_per_w // _BCHUNK

    mesh = plsc.VectorSubcoreMesh(core_axis_name="c", subcore_axis_name="s")

    @functools.partial(
        pl.kernel,
        mesh=mesh,
        out_type=jax.ShapeDtypeStruct((NB, _SPAD, _DPAD), jnp.float32),
        compiler_params=pltpu.CompilerParams(use_tc_tiling_on_sc=False),
        scratch_types=[
            pltpu.VMEM((2, _BCHUNK, SP), jnp.int32),
            pltpu.VMEM((2, _BCHUNK, _SPAD, DH), jnp.float32),
            pltpu.VMEM((2, _BCHUNK, _SPAD, DH), jnp.float32),
            pltpu.SemaphoreType.DMA,
            pltpu.SemaphoreType.DMA,
            pltpu.SemaphoreType.DMA,
            pltpu.SemaphoreType.DMA,
            pltpu.SemaphoreType.DMA,
            pltpu.SemaphoreType.DMA,
        ],
    )
    def emb(idx_hbm, wa_hbm, wb_hbm, out_hbm, idx_v, ra_v, rb_v,
            si0, si1, sg0, sg1, ss0, ss1):
        wid = lax.axis_index("s") * _NUM_CORES + lax.axis_index("c")
        base = wid * nb_per_w
        sem_i = (si0, si1)
        sem_g = (sg0, sg1)
        sem_s = (ss0, ss1)

        def idx_copy(g, b):
            return pltpu.make_async_copy(
                idx_hbm.at[pl.ds(base + g * _BCHUNK, _BCHUNK), :],
                idx_v.at[b], sem_i[b])

        def gather(b, j, table, rows):
            return pltpu.make_async_copy(
                table.at[idx_v.at[b, j, pl.ds(0, _SPAD)]],
                rows.at[b, j], sem_g[b])

        def store(g, b, rows, off):
            return pltpu.make_async_copy(
                rows.at[b],
                out_hbm.at[pl.ds(base + g * _BCHUNK, _BCHUNK), :,
                           pl.ds(off, DH)], sem_s[b])

        idx_copy(0, 0).start()
        idx_copy(1, 1).start()

        @pl.loop(0, n_chunks, step=2)
        def _(g0):
            for b in range(2):
                g = g0 + b
                idx_copy(g, b).wait()

                @pl.when(g >= 2)
                def _():
                    store(g, b, ra_v, 0).wait()
                    store(g, b, rb_v, DH).wait()

                for j in range(_BCHUNK):
                    gather(b, j, wa_hbm, ra_v).start()
                    gather(b, j, wb_hbm, rb_v).start()
                for j in range(_BCHUNK):
                    gather(b, j, wa_hbm, ra_v).wait()
                    gather(b, j, wb_hbm, rb_v).wait()

                @pl.when(g + 2 < n_chunks)
                def _():
                    idx_copy(g + 2, b).start()

                store(g, b, ra_v, 0).start()
                store(g, b, rb_v, DH).start()

        store(0, 0, ra_v, 0).wait()
        store(0, 0, rb_v, DH).wait()
        store(1, 1, ra_v, 0).wait()
        store(1, 1, rb_v, DH).wait()

    return emb(idx, wa, wb)


def kernel(input_ids, weight):
    NB, S = input_ids.shape
    V, D = weight.shape
    idx = jnp.pad(input_ids, ((0, 0), (0, 128 - S)), mode="wrap")
    wa = weight[:, :D // 2]
    wb = weight[:, D // 2:]
    out_padded = _embed(idx, wa, wb, S)
    return out_padded[:, :S, :D]


# speedup vs baseline: 1.8242x; 1.8242x over previous
import functools

import jax
import jax.numpy as jnp
from jax import lax
from jax.experimental import pallas as pl
from jax.experimental.pallas import tpu as pltpu
from jax.experimental.pallas import tpu_sc as plsc

_NUM_CORES = 2
_NUM_SUBCORES = 16
_NW = _NUM_CORES * _NUM_SUBCORES
_BCHUNK = 16
_SPAD = 56
_DPAD = 128


@functools.partial(jax.jit, static_argnums=(2,))
def _embed(idx, weight, seq):
    NB, SP = idx.shape
    V, D = weight.shape
    nb_per_w = NB // _NW
    n_chunks = nb_per_w // _BCHUNK

    mesh = plsc.VectorSubcoreMesh(core_axis_name="c", subcore_axis_name="s")

    @functools.partial(
        pl.kernel,
        mesh=mesh,
        out_type=jax.ShapeDtypeStruct((NB, _SPAD, _DPAD), jnp.float32),
        compiler_params=pltpu.CompilerParams(use_tc_tiling_on_sc=False),
        scratch_types=[
            pltpu.VMEM((2, _BCHUNK, SP), jnp.int32),
            pltpu.VMEM((2, _BCHUNK, _SPAD, D), jnp.float32),
            pltpu.SemaphoreType.DMA,
            pltpu.SemaphoreType.DMA,
            pltpu.SemaphoreType.DMA,
            pltpu.SemaphoreType.DMA,
            pltpu.SemaphoreType.DMA,
            pltpu.SemaphoreType.DMA,
        ],
    )
    def emb(idx_hbm, table_hbm, out_hbm, idx_v, rows_v, si0, si1, sg0, sg1,
            ss0, ss1):
        wid = lax.axis_index("s") * _NUM_CORES + lax.axis_index("c")
        base = wid * nb_per_w
        sem_i = (si0, si1)
        sem_g = (sg0, sg1)
        sem_s = (ss0, ss1)

        def idx_copy(g, b):
            return pltpu.make_async_copy(
                idx_hbm.at[pl.ds(base + g * _BCHUNK, _BCHUNK), :],
                idx_v.at[b], sem_i[b])

        def gather(b, j):
            return pltpu.make_async_copy(
                table_hbm.at[idx_v.at[b, j, pl.ds(0, _SPAD)]],
                rows_v.at[b, j], sem_g[b])

        def store(g, b):
            return pltpu.make_async_copy(
                rows_v.at[b],
                out_hbm.at[pl.ds(base + g * _BCHUNK, _BCHUNK), :,
                           pl.ds(0, D)], sem_s[b])

        idx_copy(0, 0).start()
        idx_copy(1, 1).start()

        @pl.loop(0, n_chunks, step=2)
        def _(g0):
            for b in range(2):
                g = g0 + b
                idx_copy(g, b).wait()

                @pl.when(g >= 2)
                def _():
                    store(g, b).wait()

                for j in range(_BCHUNK):
                    gather(b, j).start()
                for j in range(_BCHUNK):
                    gather(b, j).wait()

                @pl.when(g + 2 < n_chunks)
                def _():
                    idx_copy(g + 2, b).start()

                store(g, b).start()

        store(0, 0).wait()
        store(1, 1).wait()

    return emb(idx, weight)


def kernel(input_ids, weight):
    NB, S = input_ids.shape
    V, D = weight.shape
    idx = jnp.pad(input_ids, ((0, 0), (0, 128 - S)), mode="wrap")
    out_padded = _embed(idx, weight, S)
    return out_padded[:, :S, :D]
